# Initial kernel scaffold; baseline (speedup 1.0000x reference)
import jax
import jax.numpy as jnp
from jax.experimental import pallas as pl


def kernel(entity_embeddings, relation_embed, edge_list, edge_type, W1, W3, W_source, W_target, a0, a2_0, a_last, a2_last):
    raise NotImplementedError("write your pallas kernel here")


# speedup vs baseline: 3.8680x; 3.8680x over previous
import functools

import jax
import jax.numpy as jnp
from jax import lax
from jax.experimental import pallas as pl
from jax.experimental.pallas import tpu as pltpu
from jax.experimental.pallas import tpu_sc as plsc

N = 10000
DX = 128
DH = 64
DRL = 16
NREL = 237
NRELP = 240
NE = 160000
ALPHA = 0.2
NC = 2
NS = 16
NW = NC * NS
EPW = 5120
EPAD = EPW * NW
CH = 64
NCHUNK = EPW // CH
ZR = 48
NZC = 13
NR = NZC * ZR
NTAIL = N - NR * NS


def _elu(v):
    return jnp.where(v > 0, v, jnp.exp(v) - 1.0)


def _splat(vec16, lidx):
    return lax.gather(
        vec16, lidx[:, None],
        lax.GatherDimensionNumbers(offset_dims=(), collapsed_slice_dims=(0,),
                                   start_index_map=(0,)),
        (1,), mode=lax.GatherScatterMode.PROMISE_IN_BOUNDS)


def _tc_pre(x_ref, relp_ref, wt_ref, ws_ref,
            a0ta_ref, a1ta_ref, a0tb_ref, a1tb_ref, a2ta_ref, a2tb_ref,
            v2a_ref, v2b_ref, w1_ref, a2lt_ref, a2l_ref, w3_ref,
            ete_ref, ese_ref, u0cat_ref, u1cat_ref,
            p0a_ref, p1a_ref, p0b_ref, p1b_ref,
            rcat_ref, pra_ref, prb_ref,
            rl_ref, prl_ref, orf_ref):
    x = x_ref[...]
    f32 = jnp.float32
    ete_ref[...] = jnp.dot(x, wt_ref[...], preferred_element_type=f32)
    ese_ref[...] = jnp.dot(x, ws_ref[...], preferred_element_type=f32)
    dn = (((1,), (1,)), ((), ()))
    u0a = jnp.dot(x, a0ta_ref[...], preferred_element_type=f32)
    u1a = jnp.dot(x, a1ta_ref[...], preferred_element_type=f32)
    u0b = jnp.dot(x, a0tb_ref[...], preferred_element_type=f32)
    u1b = jnp.dot(x, a1tb_ref[...], preferred_element_type=f32)
    u0cat_ref[...] = jnp.concatenate([u0a, u0b], axis=1)
    u1cat_ref[...] = jnp.concatenate([u1a, u1b], axis=1)
    p0a_ref[...] = lax.dot_general(v2a_ref[...], u0a, dn, preferred_element_type=f32)
    p1a_ref[...] = lax.dot_general(v2a_ref[...], u1a, dn, preferred_element_type=f32)
    p0b_ref[...] = lax.dot_general(v2b_ref[...], u0b, dn, preferred_element_type=f32)
    p1b_ref[...] = lax.dot_general(v2b_ref[...], u1b, dn, preferred_element_type=f32)
    relp = relp_ref[...]
    ra = jnp.dot(relp, a2ta_ref[...], preferred_element_type=f32)
    rb = jnp.dot(relp, a2tb_ref[...], preferred_element_type=f32)
    rcat_ref[...] = jnp.concatenate([ra, rb], axis=1)
    pra_ref[...] = lax.dot_general(v2a_ref[...], ra, dn, preferred_element_type=f32)
    prb_ref[...] = lax.dot_general(v2b_ref[...], rb, dn, preferred_element_type=f32)
    o1 = jnp.dot(relp, w1_ref[...], preferred_element_type=f32)
    rl = jnp.dot(o1, a2lt_ref[...], preferred_element_type=f32)
    rl_ref[...] = rl
    prl_ref[...] = lax.dot_general(a2l_ref[...], rl, dn, preferred_element_type=f32)
    orf_ref[...] = jnp.dot(o1, w3_ref[...], preferred_element_type=f32)


def _tc_mid(u0cat_ref, tcat_ref, sa_ref, sb_ref,
            ct_ref, cs_ref, ese_ref, a0lt_ref, a1lt_ref, a2l_ref,
            u0l_ref, u1l_ref, p0l_ref, p1l_ref, tmask_ref):
    f32 = jnp.float32
    sa = sa_ref[0, :] + sa_ref[1, :]
    sb = sb_ref[0, :] + sb_ref[1, :]
    tc = tcat_ref[0] + tcat_ref[1]
    scat = jnp.concatenate(
        [jnp.broadcast_to(sa[:, None], (N, DH)),
         jnp.broadcast_to(sb[:, None], (N, DH))], axis=1)
    h = (u0cat_ref[...] * scat + tc) / (scat + 1e-12)
    x = _elu(h)
    ct = ct_ref[0, :] + ct_ref[1, :]
    cs = cs_ref[0, :] + cs_ref[1, :]
    tmask = ct > 0.5
    srcof = jnp.where(jnp.logical_and(cs > 0.5, jnp.logical_not(tmask)), 1.0, 0.0)
    x = jnp.where(srcof[:, None] > 0.5, ese_ref[...], x)
    u0l = jnp.dot(x, a0lt_ref[...], preferred_element_type=f32)
    u1l = jnp.dot(x, a1lt_ref[...], preferred_element_type=f32)
    u0l_ref[...] = u0l
    u1l_ref[...] = u1l
    dn = (((1,), (1,)), ((), ()))
    p0l_ref[...] = lax.dot_general(a2l_ref[...], u0l, dn, preferred_element_type=f32)
    p1l_ref[...] = lax.dot_general(a2l_ref[...], u1l, dn, preferred_element_type=f32)
    tmask_ref[...] = jnp.where(tmask, 1.0, 0.0)[None, :]


def _tc_fin(x_ref, u0l_ref, tl_ref, sl_ref, tmask_ref, ete_ref, out_ref):
    sl = sl_ref[0, :] + sl_ref[1, :]
    tl = tl_ref[0] + tl_ref[1]
    h = (u0l_ref[...] * sl[:, None] + tl) / (sl + 1e-12)[:, None]
    xf = _elu(h)
    tm = tmask_ref[0, :]
    out_ref[...] = jnp.where(tm[:, None] > 0.5, xf + ete_ref[...], x_ref[...])


_MESH = plsc.VectorSubcoreMesh(core_axis_name="c", subcore_axis_name="s",
                               num_cores=NC)


def _zero_vmem_2d(zb):
    def body(i, carry):
        for j in range(DX // 16):
            zb[i, pl.ds(16 * j, 16)] = jnp.zeros((16,), jnp.float32)
        return carry
    lax.fori_loop(0, ZR, body, 0)


def _zero_vmem_1d(zb1, nwords):
    def body(i, carry):
        zb1[pl.ds(16 * i, 16)] = jnp.zeros((16,), jnp.float32)
        return carry
    lax.fori_loop(0, nwords // 16, body, 0)


def _edge_pass(nh, tgt_hbm, src_hbm, et_hbm, u1_hbm, r_hbm,
               p0s, p1s, prs, t_out, s_outs, cnt_outs,
               tacc, saccs, cnt_accs,
               tgt_v, src_v, et_v, u1r, rr, g0s, g1s, grs, ws, vf,
               zb, zb1, svbuf, sem):
    c = lax.axis_index("c")
    s = lax.axis_index("s")
    wid = c * NS + s
    njh = DX // (16 * nh)

    _zero_vmem_2d(zb)
    _zero_vmem_1d(zb1, 1008)
    for k in range(NZC):
        pltpu.sync_copy(zb, tacc.at[pl.ds(NR * s + ZR * k, ZR)])

    @pl.when(s == 0)
    def _():
        pltpu.sync_copy(zb.at[pl.ds(0, NTAIL)], tacc.at[pl.ds(NR * NS, NTAIL)])

    @pl.when(s < 10)
    def _():
        accs = list(saccs) + (list(cnt_accs) if cnt_accs is not None else [])
        for a in accs:
            pltpu.sync_copy(zb1.at[pl.ds(0, 1000)], a.at[pl.ds(1000 * s, 1000)])
    plsc.subcore_barrier()

    def chunk_body(ci, carry):
        base = wid * EPW + ci * CH
        pltpu.sync_copy(tgt_hbm.at[pl.ds(base, CH)], tgt_v)
        pltpu.sync_copy(src_hbm.at[pl.ds(base, CH)], src_v)
        pltpu.sync_copy(et_hbm.at[pl.ds(base, CH)], et_v)
        copies = [pltpu.async_copy(u1_hbm.at[src_v], u1r, sem),
                  pltpu.async_copy(r_hbm.at[et_v], rr, sem)]
        for h in range(nh):
            copies.append(pltpu.async_copy(p0s[h].at[tgt_v], g0s[h], sem))
            copies.append(pltpu.async_copy(p1s[h].at[src_v], g1s[h], sem))
            copies.append(pltpu.async_copy(prs[h].at[et_v], grs[h], sem))
        for cp in copies:
            cp.wait()
        iota = lax.iota(jnp.int32, 16)
        for g in range(CH // 16):
            sl = pl.ds(16 * g, 16)
            gidx = iota + (base + 16 * g)
            valid = gidx < NE
            vf[sl] = jnp.where(valid, 1.0, 0.0)
            for h in range(nh):
                sv = g0s[h][sl] + g1s[h][sl] + grs[h][sl]
                pw = jnp.where(sv > 0, -sv, (-ALPHA) * sv)
                wv = jnp.exp(pw)
                ws[h][sl] = jnp.where(valid, wv, 0.0)

        def group_body(gi, carry2):
            gsl = pl.ds(16 * gi, 16)
            wvs = [ws[h][gsl] for h in range(nh)]
            for lane in range(16):
                i = gi * 16 + lane
                lidx = jnp.full((16,), lane, jnp.int32)
                for h in range(nh):
                    wspl = _splat(wvs[h], lidx)
                    for j in range(njh):
                        jsl = pl.ds(16 * (h * njh + j), 16)
                        u1r[i, jsl] = wspl * (u1r[i, jsl] + rr[i, jsl])
            return carry2

        lax.fori_loop(0, CH // 16, group_body, 0)
        for h in range(nh):
            pltpu.sync_copy(ws[h], saccs[h].at[tgt_v], add=True)
        pltpu.sync_copy(u1r, tacc.at[tgt_v], add=True)
        if cnt_accs is not None:
            pltpu.sync_copy(vf, cnt_accs[0].at[tgt_v], add=True)
            pltpu.sync_copy(vf, cnt_accs[1].at[src_v], add=True)
        return carry

    lax.fori_loop(0, NCHUNK, chunk_body, 0)
    plsc.subcore_barrier()

    for k in range(NZC):
        off = NR * s + ZR * k
        pltpu.sync_copy(tacc.at[pl.ds(off, ZR)], zb)
        pltpu.sync_copy(zb, t_out.at[c, pl.ds(off, ZR)])

    @pl.when(s == 0)
    def _():
        pltpu.sync_copy(tacc.at[pl.ds(NR * NS, NTAIL)], zb.at[pl.ds(0, NTAIL)])
        pltpu.sync_copy(zb.at[pl.ds(0, NTAIL)], t_out.at[c, pl.ds(NR * NS, NTAIL)])

    @pl.when(s < 10)
    def _():
        accs = list(saccs) + (list(cnt_accs) if cnt_accs is not None else [])
        outs = list(s_outs) + (list(cnt_outs) if cnt_outs is not None else [])
        for a, o in zip(accs, outs):
            pltpu.sync_copy(a.at[pl.ds(1000 * s, 1000)], svbuf)
            pltpu.sync_copy(svbuf, o.at[pl.ds(c * N + 1000 * s, 1000)])


@functools.partial(
    pl.kernel,
    out_type=[
        jax.ShapeDtypeStruct((NC, N, DX), jnp.float32),
        jax.ShapeDtypeStruct((NC * N,), jnp.float32),
        jax.ShapeDtypeStruct((NC * N,), jnp.float32),
        jax.ShapeDtypeStruct((NC * N,), jnp.float32),
        jax.ShapeDtypeStruct((NC * N,), jnp.float32),
    ],
    mesh=_MESH,
    scratch_types=[
        pltpu.VMEM_SHARED((N, DX), jnp.float32),
        pltpu.VMEM_SHARED((N,), jnp.float32),
        pltpu.VMEM_SHARED((N,), jnp.float32),
        pltpu.VMEM_SHARED((N,), jnp.float32),
        pltpu.VMEM_SHARED((N,), jnp.float32),
        pltpu.VMEM((CH,), jnp.int32),
        pltpu.VMEM((CH,), jnp.int32),
        pltpu.VMEM((CH,), jnp.int32),
        pltpu.VMEM((CH, DX), jnp.float32),
        pltpu.VMEM((CH, DX), jnp.float32),
        pltpu.VMEM((CH,), jnp.float32),
        pltpu.VMEM((CH,), jnp.float32),
        pltpu.VMEM((CH,), jnp.float32),
        pltpu.VMEM((CH,), jnp.float32),
        pltpu.VMEM((CH,), jnp.float32),
        pltpu.VMEM((CH,), jnp.float32),
        pltpu.VMEM((CH,), jnp.float32),
        pltpu.VMEM((CH,), jnp.float32),
        pltpu.VMEM((CH,), jnp.float32),
        pltpu.VMEM((ZR, DX), jnp.float32),
        pltpu.VMEM((1008,), jnp.float32),
        pltpu.VMEM((1000,), jnp.float32),
        pltpu.SemaphoreType.DMA,
    ],
)
def _sc_layer0(tgt_hbm, src_hbm, et_hbm,
               u1cat_hbm, rcat_hbm,
               p0a_hbm, p1a_hbm, pra_hbm, p0b_hbm, p1b_hbm, prb_hbm,
               t_out, sa_out, sb_out, ct_out, cs_out,
               tacc, sacc_a, sacc_b, ctacc, csacc,
               tgt_v, src_v, et_v, u1r, rr,
               g0a, g1a, gra, g0b, g1b, grb, wa, wb, vf,
               zb, zb1, svbuf, sem):
    _edge_pass(2, tgt_hbm, src_hbm, et_hbm, u1cat_hbm, rcat_hbm,
               [p0a_hbm, p0b_hbm], [p1a_hbm, p1b_hbm], [pra_hbm, prb_hbm],
               t_out, [sa_out, sb_out], [ct_out, cs_out],
               tacc, [sacc_a, sacc_b], [ctacc, csacc],
               tgt_v, src_v, et_v, u1r, rr,
               [g0a, g0b], [g1a, g1b], [gra, grb], [wa, wb], vf,
               zb, zb1, svbuf, sem)


@functools.partial(
    pl.kernel,
    out_type=[
        jax.ShapeDtypeStruct((NC, N, DX), jnp.float32),
        jax.ShapeDtypeStruct((NC * N,), jnp.float32),
    ],
    mesh=_MESH,
    scratch_types=[
        pltpu.VMEM_SHARED((N, DX), jnp.float32),
        pltpu.VMEM_SHARED((N,), jnp.float32),
        pltpu.VMEM((CH,), jnp.int32),
        pltpu.VMEM((CH,), jnp.int32),
        pltpu.VMEM((CH,), jnp.int32),
        pltpu.VMEM((CH, DX), jnp.float32),
        pltpu.VMEM((CH, DX), jnp.float32),
        pltpu.VMEM((CH,), jnp.float32),
        pltpu.VMEM((CH,), jnp.float32),
        pltpu.VMEM((CH,), jnp.float32),
        pltpu.VMEM((CH,), jnp.float32),
        pltpu.VMEM((CH,), jnp.float32),
        pltpu.VMEM((ZR, DX), jnp.float32),
        pltpu.VMEM((1008,), jnp.float32),
        pltpu.VMEM((1000,), jnp.float32),
        pltpu.SemaphoreType.DMA,
    ],
)
def _sc_final(tgt_hbm, src_hbm, et_hbm,
              u1l_hbm, rl_hbm, p0l_hbm, p1l_hbm, prl_hbm,
              tl_out, sl_out,
              tacc, sacc,
              tgt_v, src_v, et_v, u1r, rr,
              g0, g1, gr, w, vf,
              zb, zb1, svbuf, sem):
    _edge_pass(1, tgt_hbm, src_hbm, et_hbm, u1l_hbm, rl_hbm,
               [p0l_hbm], [p1l_hbm], [prl_hbm],
               tl_out, [sl_out], None,
               tacc, [sacc], None,
               tgt_v, src_v, et_v, u1r, rr,
               [g0], [g1], [gr], [w], vf,
               zb, zb1, svbuf, sem)


def kernel(entity_embeddings, relation_embed, edge_list, edge_type,
           W1, W3, W_source, W_target, a0, a2_0, a_last, a2_last):
    f32 = jnp.float32
    x = entity_embeddings.astype(f32)
    relp = jnp.zeros((NRELP, DRL), f32).at[:NREL].set(relation_embed.astype(f32))

    el = jnp.asarray(edge_list, jnp.int32)
    et = jnp.asarray(edge_type, jnp.int32)
    tgt = jnp.pad(el[0], (0, EPAD - NE))
    src = jnp.pad(el[1], (0, EPAD - NE))
    etp = jnp.pad(et, (0, EPAD - NE))

    a0 = a0.astype(f32)
    a0ta = a0[0, :, :DX].T
    a1ta = a0[0, :, DX:2 * DX].T
    a2ta = a0[0, :, 2 * DX:].T
    a0tb = a0[1, :, :DX].T
    a1tb = a0[1, :, DX:2 * DX].T
    a2tb = a0[1, :, 2 * DX:].T
    v2a = a2_0[0].astype(f32)
    v2b = a2_0[1].astype(f32)
    a_last = a_last.astype(f32)
    a0lt = a_last[:, :DX].T
    a1lt = a_last[:, DX:2 * DX].T
    a2lt = a_last[:, 2 * DX:].T
    a2l = a2_last.astype(f32)

    shp = [
        jax.ShapeDtypeStruct((N, DX), f32),
        jax.ShapeDtypeStruct((N, DX), f32),
        jax.ShapeDtypeStruct((N, DX), f32),
        jax.ShapeDtypeStruct((N, DX), f32),
        jax.ShapeDtypeStruct((1, N), f32),
        jax.ShapeDtypeStruct((1, N), f32),
        jax.ShapeDtypeStruct((1, N), f32),
        jax.ShapeDtypeStruct((1, N), f32),
        jax.ShapeDtypeStruct((NRELP, DX), f32),
        jax.ShapeDtypeStruct((1, NRELP), f32),
        jax.ShapeDtypeStruct((1, NRELP), f32),
        jax.ShapeDtypeStruct((NRELP, DX), f32),
        jax.ShapeDtypeStruct((1, NRELP), f32),
        jax.ShapeDtypeStruct((NRELP, DX), f32),
    ]
    (ete, ese, u0cat, u1cat, p0a, p1a, p0b, p1b,
     rcat, pra, prb, rl, prl, orf) = pl.pallas_call(
        _tc_pre, out_shape=shp)(
        x, relp, W_target.astype(f32), W_source.astype(f32),
        a0ta, a1ta, a0tb, a1tb, a2ta, a2tb, v2a, v2b,
        W1.astype(f32), a2lt, a2l, W3.astype(f32))

    tcat, sa, sb, ct, cs = _sc_layer0(
        tgt, src, etp, u1cat, rcat,
        p0a.reshape(N), p1a.reshape(N), pra.reshape(NRELP),
        p0b.reshape(N), p1b.reshape(N), prb.reshape(NRELP))

    u0l, u1l, p0l, p1l, tmask = pl.pallas_call(
        _tc_mid,
        out_shape=[
            jax.ShapeDtypeStruct((N, DX), f32),
            jax.ShapeDtypeStruct((N, DX), f32),
            jax.ShapeDtypeStruct((1, N), f32),
            jax.ShapeDtypeStruct((1, N), f32),
            jax.ShapeDtypeStruct((1, N), f32),
        ])(u0cat, tcat, sa.reshape(NC, N), sb.reshape(NC, N),
           ct.reshape(NC, N), cs.reshape(NC, N), ese, a0lt, a1lt, a2l)

    tl, sl = _sc_final(
        tgt, src, etp, u1l, rl,
        p0l.reshape(N), p1l.reshape(N), prl.reshape(NRELP))

    new_emb = pl.pallas_call(
        _tc_fin, out_shape=jax.ShapeDtypeStruct((N, DX), f32))(
        x, u0l, tl, sl.reshape(NC, N), tmask, ete)

    return new_emb, orf[:NREL]


# speedup vs baseline: 3.9262x; 1.0151x over previous
import functools

import jax
import jax.numpy as jnp
from jax import lax
from jax.experimental import pallas as pl
from jax.experimental.pallas import tpu as pltpu
from jax.experimental.pallas import tpu_sc as plsc

N = 10000
DX = 128
DH = 64
DRL = 16
NREL = 237
NRELP = 240
NE = 160000
ALPHA = 0.2
NC = 2
NS = 16
NW = NC * NS
EPW = 5120
EPAD = EPW * NW
CH = 64
NCHUNK = EPW // CH
ZR = 24
NZC = 26
NR = NZC * ZR
NTAIL = N - NR * NS


def _elu(v):
    return jnp.where(v > 0, v, jnp.exp(v) - 1.0)


def _splat(vec16, lidx):
    return lax.gather(
        vec16, lidx[:, None],
        lax.GatherDimensionNumbers(offset_dims=(), collapsed_slice_dims=(0,),
                                   start_index_map=(0,)),
        (1,), mode=lax.GatherScatterMode.PROMISE_IN_BOUNDS)


def _tc_pre(x_ref, relp_ref, wt_ref, ws_ref,
            a0ta_ref, a1ta_ref, a0tb_ref, a1tb_ref, a2ta_ref, a2tb_ref,
            v2a_ref, v2b_ref, w1_ref, a2lt_ref, a2l_ref, w3_ref,
            ete_ref, ese_ref, u0cat_ref, u1cat_ref,
            p0a_ref, p1a_ref, p0b_ref, p1b_ref,
            rcat_ref, pra_ref, prb_ref,
            rl_ref, prl_ref, orf_ref):
    x = x_ref[...]
    f32 = jnp.float32
    ete_ref[...] = jnp.dot(x, wt_ref[...], preferred_element_type=f32)
    ese_ref[...] = jnp.dot(x, ws_ref[...], preferred_element_type=f32)
    dn = (((1,), (1,)), ((), ()))
    u0a = jnp.dot(x, a0ta_ref[...], preferred_element_type=f32)
    u1a = jnp.dot(x, a1ta_ref[...], preferred_element_type=f32)
    u0b = jnp.dot(x, a0tb_ref[...], preferred_element_type=f32)
    u1b = jnp.dot(x, a1tb_ref[...], preferred_element_type=f32)
    u0cat_ref[...] = jnp.concatenate([u0a, u0b], axis=1)
    u1cat_ref[...] = jnp.concatenate([u1a, u1b], axis=1)
    p0a_ref[...] = lax.dot_general(v2a_ref[...], u0a, dn, preferred_element_type=f32)
    p1a_ref[...] = lax.dot_general(v2a_ref[...], u1a, dn, preferred_element_type=f32)
    p0b_ref[...] = lax.dot_general(v2b_ref[...], u0b, dn, preferred_element_type=f32)
    p1b_ref[...] = lax.dot_general(v2b_ref[...], u1b, dn, preferred_element_type=f32)
    relp = relp_ref[...]
    ra = jnp.dot(relp, a2ta_ref[...], preferred_element_type=f32)
    rb = jnp.dot(relp, a2tb_ref[...], preferred_element_type=f32)
    rcat_ref[...] = jnp.concatenate([ra, rb], axis=1)
    pra_ref[...] = lax.dot_general(v2a_ref[...], ra, dn, preferred_element_type=f32)
    prb_ref[...] = lax.dot_general(v2b_ref[...], rb, dn, preferred_element_type=f32)
    o1 = jnp.dot(relp, w1_ref[...], preferred_element_type=f32)
    rl = jnp.dot(o1, a2lt_ref[...], preferred_element_type=f32)
    rl_ref[...] = rl
    prl_ref[...] = lax.dot_general(a2l_ref[...], rl, dn, preferred_element_type=f32)
    orf_ref[...] = jnp.dot(o1, w3_ref[...], preferred_element_type=f32)


def _tc_mid(u0cat_ref, tcat_ref, sa_ref, sb_ref,
            ct_ref, cs_ref, ese_ref, a0lt_ref, a1lt_ref, a2l_ref,
            u0l_ref, u1l_ref, p0l_ref, p1l_ref, tmask_ref):
    f32 = jnp.float32
    sa = sa_ref[0, :] + sa_ref[1, :]
    sb = sb_ref[0, :] + sb_ref[1, :]
    tc = tcat_ref[0] + tcat_ref[1]
    scat = jnp.concatenate(
        [jnp.broadcast_to(sa[:, None], (N, DH)),
         jnp.broadcast_to(sb[:, None], (N, DH))], axis=1)
    h = (u0cat_ref[...] * scat + tc) / (scat + 1e-12)
    x = _elu(h)
    ct = ct_ref[0, :] + ct_ref[1, :]
    cs = cs_ref[0, :] + cs_ref[1, :]
    tmask = ct > 0.5
    srcof = jnp.where(jnp.logical_and(cs > 0.5, jnp.logical_not(tmask)), 1.0, 0.0)
    x = jnp.where(srcof[:, None] > 0.5, ese_ref[...], x)
    u0l = jnp.dot(x, a0lt_ref[...], preferred_element_type=f32)
    u1l = jnp.dot(x, a1lt_ref[...], preferred_element_type=f32)
    u0l_ref[...] = u0l
    u1l_ref[...] = u1l
    dn = (((1,), (1,)), ((), ()))
    p0l_ref[...] = lax.dot_general(a2l_ref[...], u0l, dn, preferred_element_type=f32)
    p1l_ref[...] = lax.dot_general(a2l_ref[...], u1l, dn, preferred_element_type=f32)
    tmask_ref[...] = jnp.where(tmask, 1.0, 0.0)[None, :]


def _tc_fin(x_ref, u0l_ref, tl_ref, sl_ref, tmask_ref, ete_ref, out_ref):
    sl = sl_ref[0, :] + sl_ref[1, :]
    tl = tl_ref[0] + tl_ref[1]
    h = (u0l_ref[...] * sl[:, None] + tl) / (sl + 1e-12)[:, None]
    xf = _elu(h)
    tm = tmask_ref[0, :]
    out_ref[...] = jnp.where(tm[:, None] > 0.5, xf + ete_ref[...], x_ref[...])


_MESH = plsc.VectorSubcoreMesh(core_axis_name="c", subcore_axis_name="s",
                               num_cores=NC)


def _zero_vmem_2d(zb):
    def body(i, carry):
        for j in range(DX // 16):
            zb[i, pl.ds(16 * j, 16)] = jnp.zeros((16,), jnp.float32)
        return carry
    lax.fori_loop(0, ZR, body, 0)


def _zero_vmem_1d(zb1, nwords):
    def body(i, carry):
        zb1[pl.ds(16 * i, 16)] = jnp.zeros((16,), jnp.float32)
        return carry
    lax.fori_loop(0, nwords // 16, body, 0)


class _BufSet:
    def __init__(self, nh, scr):
        self.tgt, self.src, self.et = scr[0:3]
        self.u1r, self.rr = scr[3:5]
        self.g0 = scr[5:5 + nh]
        self.g1 = scr[5 + nh:5 + 2 * nh]
        self.gr = scr[5 + 2 * nh:5 + 3 * nh]
        self.w = scr[5 + 3 * nh:5 + 4 * nh]
        self.vf = scr[5 + 4 * nh]


def _nset(nh):
    return 6 + 4 * nh


def _edge_pass(nh, tgt_hbm, src_hbm, et_hbm, u1_hbm, r_hbm,
               p0s, p1s, prs, t_out, s_outs, cnt_outs,
               tacc, saccs, cnt_accs, scr):
    c = lax.axis_index("c")
    s = lax.axis_index("s")
    wid = c * NS + s
    njh = DX // (16 * nh)
    ns = _nset(nh)
    sets = [_BufSet(nh, scr[b * ns:(b + 1) * ns]) for b in range(2)]
    zb, zb1, svbuf = scr[2 * ns:2 * ns + 3]
    isems = scr[2 * ns + 3:2 * ns + 5]
    gsems = scr[2 * ns + 5:2 * ns + 7]
    ssems = scr[2 * ns + 7:2 * ns + 9]

    _zero_vmem_2d(zb)
    _zero_vmem_1d(zb1, 1008)
    for k in range(NZC):
        pltpu.sync_copy(zb, tacc.at[pl.ds(NR * s + ZR * k, ZR)])

    @pl.when(s == 0)
    def _():
        pltpu.sync_copy(zb.at[pl.ds(0, NTAIL)], tacc.at[pl.ds(NR * NS, NTAIL)])

    @pl.when(s < 10)
    def _():
        accs = list(saccs) + (list(cnt_accs) if cnt_accs is not None else [])
        for a in accs:
            pltpu.sync_copy(zb1.at[pl.ds(0, 1000)], a.at[pl.ds(1000 * s, 1000)])
    plsc.subcore_barrier()

    def fire_idx(b, ci):
        S = sets[b]
        base = wid * EPW + ci * CH
        pltpu.async_copy(tgt_hbm.at[pl.ds(base, CH)], S.tgt, isems[b])
        pltpu.async_copy(src_hbm.at[pl.ds(base, CH)], S.src, isems[b])
        pltpu.async_copy(et_hbm.at[pl.ds(base, CH)], S.et, isems[b])

    def wait_idx(b):
        S = sets[b]
        pltpu.make_async_copy(tgt_hbm.at[pl.ds(0, CH)], S.tgt, isems[b]).wait()
        pltpu.make_async_copy(src_hbm.at[pl.ds(0, CH)], S.src, isems[b]).wait()
        pltpu.make_async_copy(et_hbm.at[pl.ds(0, CH)], S.et, isems[b]).wait()

    def fire_gathers(b):
        S = sets[b]
        pltpu.async_copy(u1_hbm.at[S.src], S.u1r, gsems[b])
        pltpu.async_copy(r_hbm.at[S.et], S.rr, gsems[b])
        for h in range(nh):
            pltpu.async_copy(p0s[h].at[S.tgt], S.g0[h], gsems[b])
            pltpu.async_copy(p1s[h].at[S.src], S.g1[h], gsems[b])
            pltpu.async_copy(prs[h].at[S.et], S.gr[h], gsems[b])

    def wait_gathers(b):
        S = sets[b]
        pltpu.make_async_copy(u1_hbm.at[S.src], S.u1r, gsems[b]).wait()
        pltpu.make_async_copy(r_hbm.at[S.et], S.rr, gsems[b]).wait()
        for h in range(nh):
            pltpu.make_async_copy(p0s[h].at[S.tgt], S.g0[h], gsems[b]).wait()
            pltpu.make_async_copy(p1s[h].at[S.src], S.g1[h], gsems[b]).wait()
            pltpu.make_async_copy(prs[h].at[S.et], S.gr[h], gsems[b]).wait()

    def fire_scatters(b):
        S = sets[b]
        for h in range(nh):
            pltpu.async_copy(S.w[h], saccs[h].at[S.tgt], ssems[b], add=True)
        pltpu.async_copy(S.u1r, tacc.at[S.tgt], ssems[b], add=True)
        if cnt_accs is not None:
            pltpu.async_copy(S.vf, cnt_accs[0].at[S.tgt], ssems[b], add=True)
            pltpu.async_copy(S.vf, cnt_accs[1].at[S.src], ssems[b], add=True)

    def wait_scatters(b):
        S = sets[b]
        for h in range(nh):
            pltpu.make_async_copy(S.w[h], saccs[h].at[S.tgt], ssems[b]).wait()
        pltpu.make_async_copy(S.u1r, tacc.at[S.tgt], ssems[b]).wait()
        if cnt_accs is not None:
            pltpu.make_async_copy(S.vf, cnt_accs[0].at[S.tgt], ssems[b]).wait()
            pltpu.make_async_copy(S.vf, cnt_accs[1].at[S.src], ssems[b]).wait()

    def compute(b, ci):
        S = sets[b]
        base = wid * EPW + ci * CH
        iota = lax.iota(jnp.int32, 16)
        for g in range(CH // 16):
            sl = pl.ds(16 * g, 16)
            gidx = iota + (base + 16 * g)
            valid = gidx < NE
            S.vf[sl] = jnp.where(valid, 1.0, 0.0)
            for h in range(nh):
                sv = S.g0[h][sl] + S.g1[h][sl] + S.gr[h][sl]
                pw = jnp.where(sv > 0, -sv, (-ALPHA) * sv)
                wv = jnp.exp(pw)
                S.w[h][sl] = jnp.where(valid, wv, 0.0)

        def group_body(gi, carry2):
            gsl = pl.ds(16 * gi, 16)
            wvs = [S.w[h][gsl] for h in range(nh)]
            for lane in range(16):
                i = gi * 16 + lane
                lidx = jnp.full((16,), lane, jnp.int32)
                for h in range(nh):
                    wspl = _splat(wvs[h], lidx)
                    for j in range(njh):
                        jsl = pl.ds(16 * (h * njh + j), 16)
                        S.u1r[i, jsl] = wspl * (S.u1r[i, jsl] + S.rr[i, jsl])
            return carry2

        lax.fori_loop(0, CH // 16, group_body, 0)

    fire_idx(0, 0)
    fire_idx(1, 1)
    wait_idx(0)
    fire_gathers(0)
    wait_idx(1)
    fire_gathers(1)

    npair = NCHUNK // 2

    def pair_body(i, carry):
        e0 = 2 * i
        wait_gathers(0)
        compute(0, e0)
        fire_scatters(0)
        wait_gathers(1)
        compute(1, e0 + 1)
        fire_scatters(1)

        @pl.when(i + 1 < npair)
        def _():
            wait_scatters(0)
            fire_idx(0, e0 + 2)
            wait_idx(0)
            fire_gathers(0)
            wait_scatters(1)
            fire_idx(1, e0 + 3)
            wait_idx(1)
            fire_gathers(1)
        return carry

    lax.fori_loop(0, npair, pair_body, 0)
    wait_scatters(0)
    wait_scatters(1)
    plsc.subcore_barrier()

    for k in range(NZC):
        off = NR * s + ZR * k
        pltpu.sync_copy(tacc.at[pl.ds(off, ZR)], zb)
        pltpu.sync_copy(zb, t_out.at[c, pl.ds(off, ZR)])

    @pl.when(s == 0)
    def _():
        pltpu.sync_copy(tacc.at[pl.ds(NR * NS, NTAIL)], zb.at[pl.ds(0, NTAIL)])
        pltpu.sync_copy(zb.at[pl.ds(0, NTAIL)], t_out.at[c, pl.ds(NR * NS, NTAIL)])

    @pl.when(s < 10)
    def _():
        accs = list(saccs) + (list(cnt_accs) if cnt_accs is not None else [])
        outs = list(s_outs) + (list(cnt_outs) if cnt_outs is not None else [])
        for a, o in zip(accs, outs):
            pltpu.sync_copy(a.at[pl.ds(1000 * s, 1000)], svbuf)
            pltpu.sync_copy(svbuf, o.at[pl.ds(c * N + 1000 * s, 1000)])


def _scratch_types(nh):
    tys = []
    for _ in range(2):
        tys += [pltpu.VMEM((CH,), jnp.int32)] * 3
        tys += [pltpu.VMEM((CH, DX), jnp.float32)] * 2
        tys += [pltpu.VMEM((CH,), jnp.float32)] * (4 * nh)
        tys += [pltpu.VMEM((CH,), jnp.float32)]
    tys += [pltpu.VMEM((ZR, DX), jnp.float32),
            pltpu.VMEM((1008,), jnp.float32),
            pltpu.VMEM((1000,), jnp.float32)]
    tys += [pltpu.SemaphoreType.DMA] * 6
    return tys


@functools.partial(
    pl.kernel,
    out_type=[
        jax.ShapeDtypeStruct((NC, N, DX), jnp.float32),
        jax.ShapeDtypeStruct((NC * N,), jnp.float32),
        jax.ShapeDtypeStruct((NC * N,), jnp.float32),
        jax.ShapeDtypeStruct((NC * N,), jnp.float32),
        jax.ShapeDtypeStruct((NC * N,), jnp.float32),
    ],
    mesh=_MESH,
    scratch_types=[
        pltpu.VMEM_SHARED((N, DX), jnp.float32),
        pltpu.VMEM_SHARED((N,), jnp.float32),
        pltpu.VMEM_SHARED((N,), jnp.float32),
        pltpu.VMEM_SHARED((N,), jnp.float32),
        pltpu.VMEM_SHARED((N,), jnp.float32),
    ] + _scratch_types(2),
)
def _sc_layer0(tgt_hbm, src_hbm, et_hbm,
               u1cat_hbm, rcat_hbm,
               p0a_hbm, p1a_hbm, pra_hbm, p0b_hbm, p1b_hbm, prb_hbm,
               t_out, sa_out, sb_out, ct_out, cs_out,
               tacc, sacc_a, sacc_b, ctacc, csacc, *scr):
    _edge_pass(2, tgt_hbm, src_hbm, et_hbm, u1cat_hbm, rcat_hbm,
               [p0a_hbm, p0b_hbm], [p1a_hbm, p1b_hbm], [pra_hbm, prb_hbm],
               t_out, [sa_out, sb_out], [ct_out, cs_out],
               tacc, [sacc_a, sacc_b], [ctacc, csacc], list(scr))


@functools.partial(
    pl.kernel,
    out_type=[
        jax.ShapeDtypeStruct((NC, N, DX), jnp.float32),
        jax.ShapeDtypeStruct((NC * N,), jnp.float32),
    ],
    mesh=_MESH,
    scratch_types=[
        pltpu.VMEM_SHARED((N, DX), jnp.float32),
        pltpu.VMEM_SHARED((N,), jnp.float32),
    ] + _scratch_types(1),
)
def _sc_final(tgt_hbm, src_hbm, et_hbm,
              u1l_hbm, rl_hbm, p0l_hbm, p1l_hbm, prl_hbm,
              tl_out, sl_out,
              tacc, sacc, *scr):
    _edge_pass(1, tgt_hbm, src_hbm, et_hbm, u1l_hbm, rl_hbm,
               [p0l_hbm], [p1l_hbm], [prl_hbm],
               tl_out, [sl_out], None,
               tacc, [sacc], None, list(scr))


def kernel(entity_embeddings, relation_embed, edge_list, edge_type,
           W1, W3, W_source, W_target, a0, a2_0, a_last, a2_last):
    f32 = jnp.float32
    x = entity_embeddings.astype(f32)
    relp = jnp.zeros((NRELP, DRL), f32).at[:NREL].set(relation_embed.astype(f32))

    el = jnp.asarray(edge_list, jnp.int32)
    et = jnp.asarray(edge_type, jnp.int32)
    tgt = jnp.pad(el[0], (0, EPAD - NE))
    src = jnp.pad(el[1], (0, EPAD - NE))
    etp = jnp.pad(et, (0, EPAD - NE))

    a0 = a0.astype(f32)
    a0ta = a0[0, :, :DX].T
    a1ta = a0[0, :, DX:2 * DX].T
    a2ta = a0[0, :, 2 * DX:].T
    a0tb = a0[1, :, :DX].T
    a1tb = a0[1, :, DX:2 * DX].T
    a2tb = a0[1, :, 2 * DX:].T
    v2a = a2_0[0].astype(f32)
    v2b = a2_0[1].astype(f32)
    a_last = a_last.astype(f32)
    a0lt = a_last[:, :DX].T
    a1lt = a_last[:, DX:2 * DX].T
    a2lt = a_last[:, 2 * DX:].T
    a2l = a2_last.astype(f32)

    shp = [
        jax.ShapeDtypeStruct((N, DX), f32),
        jax.ShapeDtypeStruct((N, DX), f32),
        jax.ShapeDtypeStruct((N, DX), f32),
        jax.ShapeDtypeStruct((N, DX), f32),
        jax.ShapeDtypeStruct((1, N), f32),
        jax.ShapeDtypeStruct((1, N), f32),
        jax.ShapeDtypeStruct((1, N), f32),
        jax.ShapeDtypeStruct((1, N), f32),
        jax.ShapeDtypeStruct((NRELP, DX), f32),
        jax.ShapeDtypeStruct((1, NRELP), f32),
        jax.ShapeDtypeStruct((1, NRELP), f32),
        jax.ShapeDtypeStruct((NRELP, DX), f32),
        jax.ShapeDtypeStruct((1, NRELP), f32),
        jax.ShapeDtypeStruct((NRELP, DX), f32),
    ]
    (ete, ese, u0cat, u1cat, p0a, p1a, p0b, p1b,
     rcat, pra, prb, rl, prl, orf) = pl.pallas_call(
        _tc_pre, out_shape=shp)(
        x, relp, W_target.astype(f32), W_source.astype(f32),
        a0ta, a1ta, a0tb, a1tb, a2ta, a2tb, v2a, v2b,
        W1.astype(f32), a2lt, a2l, W3.astype(f32))

    tcat, sa, sb, ct, cs = _sc_layer0(
        tgt, src, etp, u1cat, rcat,
        p0a.reshape(N), p1a.reshape(N), pra.reshape(NRELP),
        p0b.reshape(N), p1b.reshape(N), prb.reshape(NRELP))

    u0l, u1l, p0l, p1l, tmask = pl.pallas_call(
        _tc_mid,
        out_shape=[
            jax.ShapeDtypeStruct((N, DX), f32),
            jax.ShapeDtypeStruct((N, DX), f32),
            jax.ShapeDtypeStruct((1, N), f32),
            jax.ShapeDtypeStruct((1, N), f32),
            jax.ShapeDtypeStruct((1, N), f32),
        ])(u0cat, tcat, sa.reshape(NC, N), sb.reshape(NC, N),
           ct.reshape(NC, N), cs.reshape(NC, N), ese, a0lt, a1lt, a2l)

    tl, sl = _sc_final(
        tgt, src, etp, u1l, rl,
        p0l.reshape(N), p1l.reshape(N), prl.reshape(NRELP))

    new_emb = pl.pallas_call(
        _tc_fin, out_shape=jax.ShapeDtypeStruct((N, DX), f32))(
        x, u0l, tl, sl.reshape(NC, N), tmask, ete)

    return new_emb, orf[:NREL]
